# Initial kernel scaffold; baseline (speedup 1.0000x reference)
import jax
import jax.numpy as jnp
from jax.experimental import pallas as pl


def kernel(x, embed):
    raise NotImplementedError("write your pallas kernel here")


# speedup vs baseline: 1.5444x; 1.5444x over previous
import jax
import jax.numpy as jnp
from jax.experimental import pallas as pl


def _add_kernel(x_ref, e_ref, o_ref):
    o_ref[0] = x_ref[0] + e_ref[...]


def kernel(x, embed):
    B, T, D = x.shape
    TB = 512
    grid = (T // TB, B)
    return pl.pallas_call(
        _add_kernel,
        grid=grid,
        in_specs=[
            pl.BlockSpec((1, TB, D), lambda i, b: (b, i, 0)),
            pl.BlockSpec((TB, D), lambda i, b: (i, 0)),
        ],
        out_specs=pl.BlockSpec((1, TB, D), lambda i, b: (b, i, 0)),
        out_shape=jax.ShapeDtypeStruct((B, T, D), x.dtype),
    )(x, embed)


# speedup vs baseline: 1.5916x; 1.0305x over previous
import jax
import jax.numpy as jnp
from jax.experimental import pallas as pl


def _add_kernel(x_ref, e_ref, o_ref):
    o_ref[0] = x_ref[0] + e_ref[...]


def kernel(x, embed):
    B, T, D = x.shape
    TB = 1024
    grid = (T // TB, B)
    return pl.pallas_call(
        _add_kernel,
        grid=grid,
        in_specs=[
            pl.BlockSpec((1, TB, D), lambda i, b: (b, i, 0)),
            pl.BlockSpec((TB, D), lambda i, b: (i, 0)),
        ],
        out_specs=pl.BlockSpec((1, TB, D), lambda i, b: (b, i, 0)),
        out_shape=jax.ShapeDtypeStruct((B, T, D), x.dtype),
    )(x, embed)


# speedup vs baseline: 1.5989x; 1.0046x over previous
import jax
import jax.numpy as jnp
from jax.experimental import pallas as pl
from jax.experimental.pallas import tpu as pltpu

_CH = 512
_EB = 1024
_NBUF = 4
_D = 2048


def _chunk_base(s):
    i = s // 8
    r = s % 8
    b = r // 2
    h = r % 2
    return b * 8192 + i * _EB + h * _CH, i, h


def _x_copy(x_hbm, xbuf, xsem, s):
    base, _, _ = _chunk_base(s)
    return pltpu.make_async_copy(
        x_hbm.at[pl.ds(base, _CH), :], xbuf.at[s % _NBUF], xsem.at[s % _NBUF])


def _o_copy(o_hbm, obuf, osem, s):
    base, _, _ = _chunk_base(s)
    return pltpu.make_async_copy(
        obuf.at[s % _NBUF], o_hbm.at[pl.ds(base, _CH), :], osem.at[s % _NBUF])


def _e_copy(e_hbm, ebuf, esem, i):
    return pltpu.make_async_copy(
        e_hbm.at[pl.ds(i * _EB, _EB), :], ebuf.at[i % 2], esem.at[i % 2])


def _body(x_hbm, e_hbm, o_hbm, xbuf, ebuf, obuf, xsem, esem, osem):
    n_steps = 64

    @pl.when(pl.program_id(0) == 0)
    def _prologue():
        for c in range(_NBUF):
            _x_copy(x_hbm, xbuf, xsem, c).start()
        _e_copy(e_hbm, ebuf, esem, 0).start()
        _e_copy(e_hbm, ebuf, esem, 1).start()

    s = pl.program_id(0)
    _, i, h = _chunk_base(s)
    r = s % 8

    @pl.when(r == 0)
    def _embed_turnover():
        @pl.when(jnp.logical_and(i >= 1, i < 7))
        def _prefetch_next():
            _e_copy(e_hbm, ebuf, esem, i + 1).start()

        _e_copy(e_hbm, ebuf, esem, i).wait()

    _x_copy(x_hbm, xbuf, xsem, s).wait()

    @pl.when(s >= _NBUF)
    def _drain_out():
        _o_copy(o_hbm, obuf, osem, s - _NBUF).wait()

    obuf[s % _NBUF] = xbuf[s % _NBUF] + ebuf[i % 2, pl.ds(h * _CH, _CH), :]
    _o_copy(o_hbm, obuf, osem, s).start()

    @pl.when(s < n_steps - _NBUF)
    def _refill_x():
        _x_copy(x_hbm, xbuf, xsem, s + _NBUF).start()

    @pl.when(s == n_steps - 1)
    def _epilogue():
        for k in range(_NBUF):
            _o_copy(o_hbm, obuf, osem, s - (_NBUF - 1) + k).wait()


def kernel(x, embed):
    B, T, D = x.shape
    x2 = x.reshape(B * T, D)
    out = pl.pallas_call(
        _body,
        grid=(64,),
        in_specs=[
            pl.BlockSpec(memory_space=pltpu.HBM),
            pl.BlockSpec(memory_space=pltpu.HBM),
        ],
        out_specs=pl.BlockSpec(memory_space=pltpu.HBM),
        out_shape=jax.ShapeDtypeStruct((B * T, D), x.dtype),
        scratch_shapes=[
            pltpu.VMEM((_NBUF, _CH, _D), jnp.float32),
            pltpu.VMEM((2, _EB, _D), jnp.float32),
            pltpu.VMEM((_NBUF, _CH, _D), jnp.float32),
            pltpu.SemaphoreType.DMA((_NBUF,)),
            pltpu.SemaphoreType.DMA((2,)),
            pltpu.SemaphoreType.DMA((_NBUF,)),
        ],
        compiler_params=pltpu.CompilerParams(
            dimension_semantics=("arbitrary",)),
    )(x2, embed)
    return out.reshape(B, T, D)


# speedup vs baseline: 1.6001x; 1.0007x over previous
import jax
import jax.numpy as jnp
from jax.experimental import pallas as pl
from jax.experimental.pallas import tpu as pltpu

_CH = 512
_EB = 1024
_NBUF = 5
_D = 2048


def _chunk_base(s):
    i = s // 8
    r = s % 8
    b = r // 2
    h = r % 2
    return b * 8192 + i * _EB + h * _CH, i, h


def _x_copy(x_hbm, xbuf, xsem, s):
    base, _, _ = _chunk_base(s)
    return pltpu.make_async_copy(
        x_hbm.at[pl.ds(base, _CH), :], xbuf.at[s % _NBUF], xsem.at[s % _NBUF])


def _o_copy(o_hbm, obuf, osem, s):
    base, _, _ = _chunk_base(s)
    return pltpu.make_async_copy(
        obuf.at[s % _NBUF], o_hbm.at[pl.ds(base, _CH), :], osem.at[s % _NBUF])


def _e_copy(e_hbm, ebuf, esem, i):
    return pltpu.make_async_copy(
        e_hbm.at[pl.ds(i * _EB, _EB), :], ebuf.at[i % 2], esem.at[i % 2])


def _body(x_hbm, e_hbm, o_hbm, xbuf, ebuf, obuf, xsem, esem, osem):
    n_steps = 64

    @pl.when(pl.program_id(0) == 0)
    def _prologue():
        for c in range(_NBUF):
            _x_copy(x_hbm, xbuf, xsem, c).start()
        _e_copy(e_hbm, ebuf, esem, 0).start()
        _e_copy(e_hbm, ebuf, esem, 1).start()

    s = pl.program_id(0)
    _, i, h = _chunk_base(s)
    r = s % 8

    @pl.when(r == 0)
    def _embed_turnover():
        @pl.when(jnp.logical_and(i >= 1, i < 7))
        def _prefetch_next():
            _e_copy(e_hbm, ebuf, esem, i + 1).start()

        _e_copy(e_hbm, ebuf, esem, i).wait()

    _x_copy(x_hbm, xbuf, xsem, s).wait()

    @pl.when(s >= _NBUF)
    def _drain_out():
        _o_copy(o_hbm, obuf, osem, s - _NBUF).wait()

    obuf[s % _NBUF] = xbuf[s % _NBUF] + ebuf[i % 2, pl.ds(h * _CH, _CH), :]
    _o_copy(o_hbm, obuf, osem, s).start()

    @pl.when(s < n_steps - _NBUF)
    def _refill_x():
        _x_copy(x_hbm, xbuf, xsem, s + _NBUF).start()

    @pl.when(s == n_steps - 1)
    def _epilogue():
        for k in range(_NBUF):
            _o_copy(o_hbm, obuf, osem, s - (_NBUF - 1) + k).wait()


def kernel(x, embed):
    B, T, D = x.shape
    x2 = x.reshape(B * T, D)
    out = pl.pallas_call(
        _body,
        grid=(64,),
        in_specs=[
            pl.BlockSpec(memory_space=pltpu.HBM),
            pl.BlockSpec(memory_space=pltpu.HBM),
        ],
        out_specs=pl.BlockSpec(memory_space=pltpu.HBM),
        out_shape=jax.ShapeDtypeStruct((B * T, D), x.dtype),
        scratch_shapes=[
            pltpu.VMEM((_NBUF, _CH, _D), jnp.float32),
            pltpu.VMEM((2, _EB, _D), jnp.float32),
            pltpu.VMEM((_NBUF, _CH, _D), jnp.float32),
            pltpu.SemaphoreType.DMA((_NBUF,)),
            pltpu.SemaphoreType.DMA((2,)),
            pltpu.SemaphoreType.DMA((_NBUF,)),
        ],
        compiler_params=pltpu.CompilerParams(
            dimension_semantics=("arbitrary",)),
    )(x2, embed)
    return out.reshape(B, T, D)
